# baseline (device time: 22506 ns/iter reference)
import jax
import jax.numpy as jnp
from jax import lax
from jax.experimental import pallas as pl
from jax.experimental.pallas import tpu as pltpu

N_DEV = 8
B, Sq, D, Hq, Hkv, Dh = 2, 128, 512, 8, 2, 64
G = Hq // Hkv
SCALE = 0.125
NEG_BIG = -1e30


def kernel(x, Wq, Wo, K_ext, V_ext):
    c = K_ext.shape[1]

    def body(x_ref, wq_ref, wo_ref, k_ref, v_ref, out_ref,
             stage_ref, comm_ref, send_sems, recv_sems):
        me = lax.axis_index("i")

        barrier_sem = pltpu.get_barrier_semaphore()
        for off in range(1, N_DEV):
            pl.semaphore_signal(
                barrier_sem, inc=1,
                device_id=((me + off) % N_DEV,),
                device_id_type=pl.DeviceIdType.MESH,
            )
        pl.semaphore_wait(barrier_sem, N_DEV - 1)

        stage_ref[0] = k_ref[...].reshape(B * c, Hkv * Dh).astype(jnp.bfloat16)
        stage_ref[1] = v_ref[...].reshape(B * c, Hkv * Dh).astype(jnp.bfloat16)

        def make_rdma(off):
            return pltpu.make_async_remote_copy(
                src_ref=stage_ref,
                dst_ref=comm_ref.at[off - 1],
                send_sem=send_sems.at[off - 1],
                recv_sem=recv_sems.at[off - 1],
                device_id=((me + off) % N_DEV,),
                device_id_type=pl.DeviceIdType.MESH,
            )

        for off in range(1, N_DEV):
            make_rdma(off).start()

        q2d = lax.dot_general(
            x_ref[...].reshape(B * Sq, D).astype(jnp.bfloat16),
            wq_ref[...].astype(jnp.bfloat16),
            (((1,), (0,)), ((), ())),
            preferred_element_type=jnp.float32,
        )
        qstack = {}
        for b in range(B):
            qb = q2d[b * Sq:(b + 1) * Sq, :]
            for k in range(Hkv):
                qstack[(b, k)] = jnp.concatenate(
                    [qb[:, (G * k + g) * Dh:(G * k + g + 1) * Dh]
                     for g in range(G)], axis=0,
                ).astype(jnp.bfloat16)

        m = {key: jnp.full((G * Sq, 1), NEG_BIG, jnp.float32) for key in qstack}
        l = {key: jnp.zeros((G * Sq, 1), jnp.float32) for key in qstack}
        acc = {key: jnp.zeros((G * Sq, Dh), jnp.float32) for key in qstack}

        def fold(chunk):
            for b in range(B):
                for k in range(Hkv):
                    kc = chunk[0, b * c:(b + 1) * c, k * Dh:(k + 1) * Dh]
                    vc = chunk[1, b * c:(b + 1) * c, k * Dh:(k + 1) * Dh]
                    s = lax.dot_general(
                        qstack[(b, k)], kc,
                        (((1,), (1,)), ((), ())),
                        preferred_element_type=jnp.float32,
                    ) * SCALE
                    mj = jnp.max(s, axis=-1, keepdims=True)
                    m_new = jnp.maximum(m[(b, k)], mj)
                    alpha = jnp.exp(m[(b, k)] - m_new)
                    p = jnp.exp(s - m_new)
                    l[(b, k)] = l[(b, k)] * alpha + jnp.sum(p, -1, keepdims=True)
                    acc[(b, k)] = acc[(b, k)] * alpha + lax.dot_general(
                        p.astype(jnp.bfloat16), vc,
                        (((1,), (0,)), ((), ())),
                        preferred_element_type=jnp.float32,
                    )
                    m[(b, k)] = m_new

        fold(stage_ref[...])
        for s in range(N_DEV - 1):
            make_rdma(s + 1).wait_recv()
            fold(comm_ref[s])

        wo = wo_ref[...].astype(jnp.bfloat16)
        for b in range(B):
            cols = []
            for k in range(Hkv):
                o = acc[(b, k)] / l[(b, k)]
                for g in range(G):
                    cols.append(o[g * Sq:(g + 1) * Sq, :])
            attn = jnp.concatenate(cols, axis=1).astype(jnp.bfloat16)
            out_ref[b] = lax.dot_general(
                attn, wo, (((1,), (0,)), ((), ())),
                preferred_element_type=jnp.float32,
            )

        for off in range(1, N_DEV):
            make_rdma(off).wait_send()

    return pl.pallas_call(
        body,
        out_shape=jax.ShapeDtypeStruct((B, Sq, D), jnp.float32),
        in_specs=[pl.BlockSpec(memory_space=pltpu.VMEM)] * 5,
        out_specs=pl.BlockSpec(memory_space=pltpu.VMEM),
        scratch_shapes=[
            pltpu.VMEM((2, B * c, Hkv * Dh), jnp.bfloat16),
            pltpu.VMEM((N_DEV - 1, 2, B * c, Hkv * Dh), jnp.bfloat16),
            pltpu.SemaphoreType.DMA((N_DEV - 1,)),
            pltpu.SemaphoreType.DMA((N_DEV - 1,)),
        ],
        compiler_params=pltpu.CompilerParams(collective_id=0),
    )(x, Wq, Wo, K_ext, V_ext)


# device time: 21781 ns/iter; 1.0333x vs baseline; 1.0333x over previous
import jax
import jax.numpy as jnp
from jax import lax
from jax.experimental import pallas as pl
from jax.experimental.pallas import tpu as pltpu

N_DEV = 8
B, Sq, D, Hq, Hkv, Dh = 2, 128, 512, 8, 2, 64
G = Hq // Hkv
SCALE = 0.125
NEG_BIG = -1e30


def kernel(x, Wq, Wo, K_ext, V_ext):
    c = K_ext.shape[1]

    def body(x_ref, wq_ref, wo_ref, k_ref, v_ref, out_ref,
             stage_ref, comm_ref, send_sems, recv_sems):
        me = lax.axis_index("i")

        barrier_sem = pltpu.get_barrier_semaphore()
        for off in range(1, N_DEV):
            pl.semaphore_signal(
                barrier_sem, inc=1,
                device_id=((me + off) % N_DEV,),
                device_id_type=pl.DeviceIdType.MESH,
            )
        pl.semaphore_wait(barrier_sem, N_DEV - 1)

        stage_ref[0] = k_ref[...].reshape(B * c, Hkv * Dh).astype(jnp.bfloat16)
        stage_ref[1] = v_ref[...].reshape(B * c, Hkv * Dh).astype(jnp.bfloat16)

        def make_rdma(off):
            return pltpu.make_async_remote_copy(
                src_ref=stage_ref,
                dst_ref=comm_ref.at[off - 1],
                send_sem=send_sems.at[off - 1],
                recv_sem=recv_sems.at[off - 1],
                device_id=((me + off) % N_DEV,),
                device_id_type=pl.DeviceIdType.MESH,
            )

        for off in range(1, N_DEV):
            make_rdma(off).start()

        q2d = lax.dot_general(
            x_ref[...].reshape(B * Sq, D).astype(jnp.bfloat16),
            wq_ref[...].astype(jnp.bfloat16),
            (((1,), (0,)), ((), ())),
            preferred_element_type=jnp.float32,
        )
        qstack = {}
        for b in range(B):
            qb = q2d[b * Sq:(b + 1) * Sq, :]
            for k in range(Hkv):
                qstack[(b, k)] = jnp.concatenate(
                    [qb[:, (G * k + g) * Dh:(G * k + g + 1) * Dh]
                     for g in range(G)], axis=0,
                ).astype(jnp.bfloat16)

        chunks = [stage_ref[...]]
        for s in range(N_DEV - 1):
            make_rdma(s + 1).wait_recv()
            chunks.append(comm_ref[s])

        wo = wo_ref[...].astype(jnp.bfloat16)
        for b in range(B):
            cols = []
            for k in range(Hkv):
                kall = jnp.concatenate(
                    [ch[0, b * c:(b + 1) * c, k * Dh:(k + 1) * Dh]
                     for ch in chunks], axis=0)
                vall = jnp.concatenate(
                    [ch[1, b * c:(b + 1) * c, k * Dh:(k + 1) * Dh]
                     for ch in chunks], axis=0)
                s = lax.dot_general(
                    qstack[(b, k)], kall,
                    (((1,), (1,)), ((), ())),
                    preferred_element_type=jnp.float32,
                ) * SCALE
                mx = jnp.max(s, axis=-1, keepdims=True)
                p = jnp.exp(s - mx)
                lsum = jnp.sum(p, axis=-1, keepdims=True)
                o = lax.dot_general(
                    p.astype(jnp.bfloat16), vall,
                    (((1,), (0,)), ((), ())),
                    preferred_element_type=jnp.float32,
                ) / lsum
                for g in range(G):
                    cols.append(o[g * Sq:(g + 1) * Sq, :])
            attn = jnp.concatenate(cols, axis=1).astype(jnp.bfloat16)
            out_ref[b] = lax.dot_general(
                attn, wo, (((1,), (0,)), ((), ())),
                preferred_element_type=jnp.float32,
            )

        for off in range(1, N_DEV):
            make_rdma(off).wait_send()

    return pl.pallas_call(
        body,
        out_shape=jax.ShapeDtypeStruct((B, Sq, D), jnp.float32),
        in_specs=[pl.BlockSpec(memory_space=pltpu.VMEM)] * 5,
        out_specs=pl.BlockSpec(memory_space=pltpu.VMEM),
        scratch_shapes=[
            pltpu.VMEM((2, B * c, Hkv * Dh), jnp.bfloat16),
            pltpu.VMEM((N_DEV - 1, 2, B * c, Hkv * Dh), jnp.bfloat16),
            pltpu.SemaphoreType.DMA((N_DEV - 1,)),
            pltpu.SemaphoreType.DMA((N_DEV - 1,)),
        ],
        compiler_params=pltpu.CompilerParams(collective_id=0),
    )(x, Wq, Wo, K_ext, V_ext)
